# baseline (speedup 1.0000x reference)
import jax
import jax.numpy as jnp
from jax.experimental import pallas as pl
from jax.experimental.pallas import tpu as pltpu

NUM_BINS = 32
INV_BIN_WIDTH = 32.0


def _bin_body(x_ref, o_ref):
    x = x_ref[...]
    scaled = jnp.clip(x, 0.0, 1.0) * INV_BIN_WIDTH
    o_ref[...] = jnp.minimum(scaled.astype(jnp.int32), NUM_BINS - 1)


def kernel(values):
    M, N = values.shape
    BM = 256
    return pl.pallas_call(
        _bin_body,
        grid=(M // BM,),
        in_specs=[pl.BlockSpec((BM, N), lambda i: (i, 0))],
        out_specs=pl.BlockSpec((BM, N), lambda i: (i, 0)),
        out_shape=jax.ShapeDtypeStruct((M, N), jnp.int32),
        compiler_params=pltpu.CompilerParams(
            dimension_semantics=("parallel",),
        ),
    )(values)
